# Initial kernel scaffold; baseline (speedup 1.0000x reference)
import jax
import jax.numpy as jnp
from jax.experimental import pallas as pl


def kernel(pair_states, macro_state, W1, b1, ln1_g, ln1_b, W2, b2, po_W, po_b, po_g, po_beta, pair_mem_keys, pair_mem_vals, macro_mem_keys, macro_mem_vals):
    raise NotImplementedError("write your pallas kernel here")


# speedup vs baseline: 1.3661x; 1.3661x over previous
import functools

import jax
import jax.numpy as jnp
from jax import lax
from jax.experimental import pallas as pl

B = 1024
P = 32
D = 64
M = 128
S = 4096
V = 2048

SBLK = 512
KBLK = 512
BBLK = 256

_F32 = jnp.float32


def _dot(a, b, dims):
    return lax.dot_general(a, b, (dims, ((), ())), preferred_element_type=_F32)


def _stats_body(q_ref, k_ref, q_out_ref, m_ref, l_ref, w_ref, slot_ref, *,
                kd, mean_pairs):
    q = q_ref[...]
    if mean_pairs:
        q = jnp.mean(q, axis=1)
    keys = k_ref[...]
    scores = _dot(q, keys, (((1,), (1,)))) * (1.0 / (kd ** 0.5))
    m = jnp.max(scores, axis=1)
    l = jnp.sum(jnp.exp(scores - m[:, None]), axis=1)
    idx = lax.broadcasted_iota(jnp.int32, scores.shape, 1)
    slot = jnp.min(jnp.where(scores == m[:, None], idx, S), axis=1)
    surprise = 1.0 - 1.0 / l
    w = 0.1 * jax.nn.sigmoid(surprise)
    q_out_ref[...] = q
    m_ref[...] = m
    l_ref[...] = l
    w_ref[...] = w
    slot_ref[...] = slot.astype(jnp.int32)


def _stats(q_in, keys, kd, mean_pairs):
    qd = q_in.shape[-1] if not mean_pairs else q_in.shape[-1]
    return pl.pallas_call(
        functools.partial(_stats_body, kd=kd, mean_pairs=mean_pairs),
        out_shape=(
            jax.ShapeDtypeStruct((B, qd), _F32),
            jax.ShapeDtypeStruct((B,), _F32),
            jax.ShapeDtypeStruct((B,), _F32),
            jax.ShapeDtypeStruct((B,), _F32),
            jax.ShapeDtypeStruct((B,), jnp.int32),
        ),
    )(q_in, keys)


def _rw_body(q_ref, keys_ref, vals_ref, actual_ref, m_ref, l_ref, w_ref,
             slot_ref, retr_ref, nk_ref, nv_ref, *, kd, nblk):
    j = pl.program_id(0)
    q = q_ref[...]
    keys = keys_ref[...]
    vals = vals_ref[...]
    scores_t = _dot(keys, q, (((1,), (1,)))) * (1.0 / (kd ** 0.5))
    m = m_ref[...]
    l = l_ref[...]
    attn_t = jnp.exp(scores_t - m[None, :]) / l[None, :]

    part = _dot(attn_t, vals, (((0,), (0,))))

    @pl.when(j == 0)
    def _():
        retr_ref[...] = jnp.zeros_like(retr_ref)

    retr_ref[...] += part

    rows = j * SBLK + lax.broadcasted_iota(jnp.int32, (SBLK, 1), 0)
    slot = slot_ref[...]
    w = w_ref[...]
    mw = jnp.where(rows == slot[None, :], w[None, :], 0.0)
    denom = jnp.sum(mw, axis=1)
    numer_v = _dot(mw, actual_ref[...], (((1,), (0,))))
    nv_ref[...] = vals * (1.0 - denom)[:, None] + numer_v
    numer_k = _dot(mw, q, (((1,), (0,))))
    nk_ref[...] = keys * (1.0 - denom)[:, None] + numer_k


def _read_write(q, keys, vals, actual, m, l, w, slot, kd):
    nblk = S // SBLK
    full1d = pl.BlockSpec((B,), lambda j: (0,))
    return pl.pallas_call(
        functools.partial(_rw_body, kd=kd, nblk=nblk),
        grid=(nblk,),
        in_specs=[
            pl.BlockSpec((B, kd), lambda j: (0, 0)),
            pl.BlockSpec((SBLK, kd), lambda j: (j, 0)),
            pl.BlockSpec((SBLK, V), lambda j: (j, 0)),
            pl.BlockSpec((B, V), lambda j: (0, 0)),
            full1d, full1d, full1d, full1d,
        ],
        out_specs=(
            pl.BlockSpec((B, V), lambda j: (0, 0)),
            pl.BlockSpec((SBLK, kd), lambda j: (j, 0)),
            pl.BlockSpec((SBLK, V), lambda j: (j, 0)),
        ),
        out_shape=(
            jax.ShapeDtypeStruct((B, V), _F32),
            jax.ShapeDtypeStruct((S, kd), _F32),
            jax.ShapeDtypeStruct((S, V), _F32),
        ),
    )(q, keys, vals, actual, m, l, w, slot)


def _h_body(pc_ref, mc_ref, w1_ref, b1_ref, g_ref, beta_ref, h_ref, *, nk):
    k = pl.program_id(0)
    wblk = w1_ref[...]

    @pl.when(k == 0)
    def _():
        h_ref[...] = jnp.zeros_like(h_ref)

    half = nk // 2

    @pl.when(k < half)
    def _():
        h_ref[...] += _dot(pc_ref[...], wblk, (((1,), (0,))))

    @pl.when(k >= half)
    def _():
        h_ref[...] += _dot(mc_ref[...], wblk, (((1,), (0,))))

    @pl.when(k == nk - 1)
    def _():
        h = h_ref[...] + b1_ref[...][None, :]
        mu = jnp.mean(h, axis=1, keepdims=True)
        var = jnp.mean((h - mu) ** 2, axis=1, keepdims=True)
        h = (h - mu) / jnp.sqrt(var + 1e-5) * g_ref[...][None, :] \
            + beta_ref[...][None, :]
        h_ref[...] = 0.5 * h * (1.0 + lax.erf(h * (0.5 ** 0.5)))


def _mlp_h(pair_corr, macro_corr, w1, b1, g, beta):
    nk = (2 * V) // KBLK
    half = nk // 2
    fullv = pl.BlockSpec((V,), lambda k: (0,))
    return pl.pallas_call(
        functools.partial(_h_body, nk=nk),
        grid=(nk,),
        in_specs=[
            pl.BlockSpec((B, KBLK), lambda k: (0, jnp.minimum(k, half - 1))),
            pl.BlockSpec((B, KBLK),
                         lambda k: (0, jnp.maximum(k - half, 0))),
            pl.BlockSpec((KBLK, V), lambda k: (k, 0)),
            fullv, fullv, fullv,
        ],
        out_specs=pl.BlockSpec((B, V), lambda k: (0, 0)),
        out_shape=jax.ShapeDtypeStruct((B, V), _F32),
    )(pair_corr, macro_corr, w1, b1, g, beta)


def _o_body(h_ref, w2_ref, b2_ref, o_ref, *, nk):
    k = pl.program_id(0)

    @pl.when(k == 0)
    def _():
        o_ref[...] = jnp.zeros_like(o_ref)

    o_ref[...] += _dot(h_ref[...], w2_ref[...], (((1,), (0,))))

    @pl.when(k == nk - 1)
    def _():
        o_ref[...] += b2_ref[...][None, :]


def _mlp_o(h, w2, b2):
    nk = V // KBLK
    return pl.pallas_call(
        functools.partial(_o_body, nk=nk),
        grid=(nk,),
        in_specs=[
            pl.BlockSpec((B, KBLK), lambda k: (0, k)),
            pl.BlockSpec((KBLK, V), lambda k: (k, 0)),
            pl.BlockSpec((V,), lambda k: (0,)),
        ],
        out_specs=pl.BlockSpec((B, V), lambda k: (0, 0)),
        out_shape=jax.ShapeDtypeStruct((B, V), _F32),
    )(h, w2, b2)


def _head_body(ps_ref, f_ref, pw_ref, pb_ref, pg_ref, pbeta_ref, out_ref):
    ps = ps_ref[...]
    f = f_ref[...]
    pw = pw_ref[...]
    pb = pb_ref[...]
    pg = pg_ref[...]
    pbeta = pbeta_ref[...]
    for p in range(P):
        x = jnp.concatenate([ps[:, p, :], f[:, p, :]], axis=-1)
        e = _dot(x, pw[p], (((1,), (0,)))) + pb[p][None, :]
        mu = jnp.mean(e, axis=1, keepdims=True)
        var = jnp.mean((e - mu) ** 2, axis=1, keepdims=True)
        e = (e - mu) / jnp.sqrt(var + 1e-5) * pg[p][None, :] \
            + pbeta[p][None, :]
        out_ref[:, p, :] = e


def _heads(pair_states, fused, po_W, po_b, po_g, po_beta):
    nb = B // BBLK
    return pl.pallas_call(
        _head_body,
        grid=(nb,),
        in_specs=[
            pl.BlockSpec((BBLK, P, D), lambda i: (i, 0, 0)),
            pl.BlockSpec((BBLK, P, D), lambda i: (i, 0, 0)),
            pl.BlockSpec((P, 2 * D, D), lambda i: (0, 0, 0)),
            pl.BlockSpec((P, D), lambda i: (0, 0)),
            pl.BlockSpec((P, D), lambda i: (0, 0)),
            pl.BlockSpec((P, D), lambda i: (0, 0)),
        ],
        out_specs=pl.BlockSpec((BBLK, P, D), lambda i: (i, 0, 0)),
        out_shape=jax.ShapeDtypeStruct((B, P, D), _F32),
    )(pair_states, fused, po_W, po_b, po_g, po_beta)


def kernel(pair_states, macro_state, W1, b1, ln1_g, ln1_b, W2, b2, po_W,
           po_b, po_g, po_beta, pair_mem_keys, pair_mem_vals,
           macro_mem_keys, macro_mem_vals):
    actual = pair_states.reshape(B, V)

    pq, pm, plse, pw, pslot = _stats(pair_states, pair_mem_keys, D, True)
    mq, mm, mlse, mw_, mslot = _stats(macro_state, macro_mem_keys, M, False)

    pair_corr, new_pk, new_pv = _read_write(
        pq, pair_mem_keys, pair_mem_vals, actual, pm, plse, pw, pslot, D)
    macro_corr, new_mk, new_mv = _read_write(
        mq, macro_mem_keys, macro_mem_vals, actual, mm, mlse, mw_, mslot, M)

    h = _mlp_h(pair_corr, macro_corr, W1, b1, ln1_g, ln1_b)
    fused = _mlp_o(h, W2, b2).reshape(B, P, D)

    enriched = _heads(pair_states, fused, po_W, po_b, po_g, po_beta)
    return (enriched, new_pk, new_pv, new_mk, new_mv)


# speedup vs baseline: 1.3869x; 1.0152x over previous
import functools

import jax
import jax.numpy as jnp
from jax import lax
from jax.experimental import pallas as pl

B = 1024
P = 32
D = 64
M = 128
S = 4096
V = 2048

SBLK = 512
KBLK = 512
BBLK = 256

_F32 = jnp.float32


def _dot(a, b, dims):
    return lax.dot_general(a, b, (dims, ((), ())), preferred_element_type=_F32)


def _bdot(a, b, dims):
    return lax.dot_general(a.astype(jnp.bfloat16), b.astype(jnp.bfloat16),
                           (dims, ((), ())), preferred_element_type=_F32)


def _stats_body(q_ref, k_ref, q_out_ref, m_ref, l_ref, w_ref, slot_ref, *,
                kd, mean_pairs):
    q = q_ref[...]
    if mean_pairs:
        q = jnp.mean(q, axis=1)
    keys = k_ref[...]
    scores = _dot(q, keys, (((1,), (1,)))) * (1.0 / (kd ** 0.5))
    m = jnp.max(scores, axis=1)
    l = jnp.sum(jnp.exp(scores - m[:, None]), axis=1)
    idx = lax.broadcasted_iota(jnp.int32, scores.shape, 1)
    slot = jnp.min(jnp.where(scores == m[:, None], idx, S), axis=1)
    surprise = 1.0 - 1.0 / l
    w = 0.1 * jax.nn.sigmoid(surprise)
    q_out_ref[...] = q
    m_ref[...] = m
    l_ref[...] = l
    w_ref[...] = w
    slot_ref[...] = slot.astype(jnp.int32)


def _stats(q_in, keys, kd, mean_pairs):
    qd = q_in.shape[-1] if not mean_pairs else q_in.shape[-1]
    return pl.pallas_call(
        functools.partial(_stats_body, kd=kd, mean_pairs=mean_pairs),
        out_shape=(
            jax.ShapeDtypeStruct((B, qd), _F32),
            jax.ShapeDtypeStruct((B,), _F32),
            jax.ShapeDtypeStruct((B,), _F32),
            jax.ShapeDtypeStruct((B,), _F32),
            jax.ShapeDtypeStruct((B,), jnp.int32),
        ),
    )(q_in, keys)


def _rw_body(q_ref, keys_ref, vals_ref, actual_ref, m_ref, l_ref, w_ref,
             slot_ref, retr_ref, nk_ref, nv_ref, *, kd, nblk):
    j = pl.program_id(0)
    q = q_ref[...]
    keys = keys_ref[...]
    vals = vals_ref[...]
    scores_t = _dot(keys, q, (((1,), (1,)))) * (1.0 / (kd ** 0.5))
    m = m_ref[...]
    l = l_ref[...]
    attn_t = jnp.exp(scores_t - m[None, :]) / l[None, :]

    part = _bdot(attn_t, vals, (((0,), (0,))))

    @pl.when(j == 0)
    def _():
        retr_ref[...] = jnp.zeros_like(retr_ref)

    retr_ref[...] += part

    rows = j * SBLK + lax.broadcasted_iota(jnp.int32, (SBLK, 1), 0)
    slot = slot_ref[...]
    w = w_ref[...]
    mw = jnp.where(rows == slot[None, :], w[None, :], 0.0)
    denom = jnp.sum(mw, axis=1)
    numer_v = _bdot(mw, actual_ref[...], (((1,), (0,))))
    nv_ref[...] = vals * (1.0 - denom)[:, None] + numer_v
    numer_k = _dot(mw, q, (((1,), (0,))))
    nk_ref[...] = keys * (1.0 - denom)[:, None] + numer_k


def _read_write(q, keys, vals, actual, m, l, w, slot, kd):
    nblk = S // SBLK
    full1d = pl.BlockSpec((B,), lambda j: (0,))
    return pl.pallas_call(
        functools.partial(_rw_body, kd=kd, nblk=nblk),
        grid=(nblk,),
        in_specs=[
            pl.BlockSpec((B, kd), lambda j: (0, 0)),
            pl.BlockSpec((SBLK, kd), lambda j: (j, 0)),
            pl.BlockSpec((SBLK, V), lambda j: (j, 0)),
            pl.BlockSpec((B, V), lambda j: (0, 0)),
            full1d, full1d, full1d, full1d,
        ],
        out_specs=(
            pl.BlockSpec((B, V), lambda j: (0, 0)),
            pl.BlockSpec((SBLK, kd), lambda j: (j, 0)),
            pl.BlockSpec((SBLK, V), lambda j: (j, 0)),
        ),
        out_shape=(
            jax.ShapeDtypeStruct((B, V), _F32),
            jax.ShapeDtypeStruct((S, kd), _F32),
            jax.ShapeDtypeStruct((S, V), _F32),
        ),
    )(q, keys, vals, actual, m, l, w, slot)


def _h_body(pc_ref, mc_ref, w1_ref, b1_ref, g_ref, beta_ref, h_ref, *, nk):
    k = pl.program_id(0)
    wblk = w1_ref[...]

    @pl.when(k == 0)
    def _():
        h_ref[...] = jnp.zeros_like(h_ref)

    half = nk // 2

    @pl.when(k < half)
    def _():
        h_ref[...] += _bdot(pc_ref[...], wblk, (((1,), (0,))))

    @pl.when(k >= half)
    def _():
        h_ref[...] += _bdot(mc_ref[...], wblk, (((1,), (0,))))

    @pl.when(k == nk - 1)
    def _():
        h = h_ref[...] + b1_ref[...][None, :]
        mu = jnp.mean(h, axis=1, keepdims=True)
        var = jnp.mean((h - mu) ** 2, axis=1, keepdims=True)
        h = (h - mu) / jnp.sqrt(var + 1e-5) * g_ref[...][None, :] \
            + beta_ref[...][None, :]
        h_ref[...] = 0.5 * h * (1.0 + lax.erf(h * (0.5 ** 0.5)))


def _mlp_h(pair_corr, macro_corr, w1, b1, g, beta):
    nk = (2 * V) // KBLK
    half = nk // 2
    fullv = pl.BlockSpec((V,), lambda k: (0,))
    return pl.pallas_call(
        functools.partial(_h_body, nk=nk),
        grid=(nk,),
        in_specs=[
            pl.BlockSpec((B, KBLK), lambda k: (0, jnp.minimum(k, half - 1))),
            pl.BlockSpec((B, KBLK),
                         lambda k: (0, jnp.maximum(k - half, 0))),
            pl.BlockSpec((KBLK, V), lambda k: (k, 0)),
            fullv, fullv, fullv,
        ],
        out_specs=pl.BlockSpec((B, V), lambda k: (0, 0)),
        out_shape=jax.ShapeDtypeStruct((B, V), _F32),
    )(pair_corr, macro_corr, w1, b1, g, beta)


def _o_body(h_ref, w2_ref, b2_ref, o_ref, *, nk):
    k = pl.program_id(0)

    @pl.when(k == 0)
    def _():
        o_ref[...] = jnp.zeros_like(o_ref)

    o_ref[...] += _bdot(h_ref[...], w2_ref[...], (((1,), (0,))))

    @pl.when(k == nk - 1)
    def _():
        o_ref[...] += b2_ref[...][None, :]


def _mlp_o(h, w2, b2):
    nk = V // KBLK
    return pl.pallas_call(
        functools.partial(_o_body, nk=nk),
        grid=(nk,),
        in_specs=[
            pl.BlockSpec((B, KBLK), lambda k: (0, k)),
            pl.BlockSpec((KBLK, V), lambda k: (k, 0)),
            pl.BlockSpec((V,), lambda k: (0,)),
        ],
        out_specs=pl.BlockSpec((B, V), lambda k: (0, 0)),
        out_shape=jax.ShapeDtypeStruct((B, V), _F32),
    )(h, w2, b2)


def _head_body(ps_ref, f_ref, pw_ref, pb_ref, pg_ref, pbeta_ref, out_ref):
    ps = ps_ref[...]
    f = f_ref[...]
    pw = pw_ref[...]
    pb = pb_ref[...]
    pg = pg_ref[...]
    pbeta = pbeta_ref[...]
    for p in range(P):
        x = jnp.concatenate([ps[:, p, :], f[:, p, :]], axis=-1)
        e = _dot(x, pw[p], (((1,), (0,)))) + pb[p][None, :]
        mu = jnp.mean(e, axis=1, keepdims=True)
        var = jnp.mean((e - mu) ** 2, axis=1, keepdims=True)
        e = (e - mu) / jnp.sqrt(var + 1e-5) * pg[p][None, :] \
            + pbeta[p][None, :]
        out_ref[:, p, :] = e


def _heads(pair_states, fused, po_W, po_b, po_g, po_beta):
    nb = B // BBLK
    return pl.pallas_call(
        _head_body,
        grid=(nb,),
        in_specs=[
            pl.BlockSpec((BBLK, P, D), lambda i: (i, 0, 0)),
            pl.BlockSpec((BBLK, P, D), lambda i: (i, 0, 0)),
            pl.BlockSpec((P, 2 * D, D), lambda i: (0, 0, 0)),
            pl.BlockSpec((P, D), lambda i: (0, 0)),
            pl.BlockSpec((P, D), lambda i: (0, 0)),
            pl.BlockSpec((P, D), lambda i: (0, 0)),
        ],
        out_specs=pl.BlockSpec((BBLK, P, D), lambda i: (i, 0, 0)),
        out_shape=jax.ShapeDtypeStruct((B, P, D), _F32),
    )(pair_states, fused, po_W, po_b, po_g, po_beta)


def kernel(pair_states, macro_state, W1, b1, ln1_g, ln1_b, W2, b2, po_W,
           po_b, po_g, po_beta, pair_mem_keys, pair_mem_vals,
           macro_mem_keys, macro_mem_vals):
    actual = pair_states.reshape(B, V)

    pq, pm, plse, pw, pslot = _stats(pair_states, pair_mem_keys, D, True)
    mq, mm, mlse, mw_, mslot = _stats(macro_state, macro_mem_keys, M, False)

    pair_corr, new_pk, new_pv = _read_write(
        pq, pair_mem_keys, pair_mem_vals, actual, pm, plse, pw, pslot, D)
    macro_corr, new_mk, new_mv = _read_write(
        mq, macro_mem_keys, macro_mem_vals, actual, mm, mlse, mw_, mslot, M)

    h = _mlp_h(pair_corr, macro_corr, W1, b1, ln1_g, ln1_b)
    fused = _mlp_o(h, W2, b2).reshape(B, P, D)

    enriched = _heads(pair_states, fused, po_W, po_b, po_g, po_beta)
    return (enriched, new_pk, new_pv, new_mk, new_mv)


# speedup vs baseline: 1.6967x; 1.2234x over previous
import functools

import jax
import jax.numpy as jnp
from jax import lax
from jax.experimental import pallas as pl

B = 1024
P = 32
D = 64
M = 128
S = 4096
V = 2048

SBLK = 512
KBLK = 512
BBLK = 256

_F32 = jnp.float32


def _dot(a, b, dims):
    return lax.dot_general(a, b, (dims, ((), ())), preferred_element_type=_F32)


def _bdot(a, b, dims):
    return lax.dot_general(a.astype(jnp.bfloat16), b.astype(jnp.bfloat16),
                           (dims, ((), ())), preferred_element_type=_F32)


def _stats_body(q_ref, k_ref, q_out_ref, m_ref, l_ref, w_ref, slot_ref, *,
                kd, mean_pairs):
    q = q_ref[...]
    if mean_pairs:
        q = jnp.mean(q, axis=1)
    keys = k_ref[...]
    scores = _dot(q, keys, (((1,), (1,)))) * (1.0 / (kd ** 0.5))
    m = jnp.max(scores, axis=1)
    l = jnp.sum(jnp.exp(scores - m[:, None]), axis=1)
    idx = lax.broadcasted_iota(jnp.int32, scores.shape, 1)
    slot = jnp.min(jnp.where(scores == m[:, None], idx, S), axis=1)
    surprise = 1.0 - 1.0 / l
    w = 0.1 * jax.nn.sigmoid(surprise)
    q_out_ref[...] = q
    m_ref[...] = m
    l_ref[...] = l
    w_ref[...] = w
    slot_ref[...] = slot.astype(jnp.int32)


def _stats(q_in, keys, kd, mean_pairs):
    qd = q_in.shape[-1] if not mean_pairs else q_in.shape[-1]
    return pl.pallas_call(
        functools.partial(_stats_body, kd=kd, mean_pairs=mean_pairs),
        out_shape=(
            jax.ShapeDtypeStruct((B, qd), _F32),
            jax.ShapeDtypeStruct((B,), _F32),
            jax.ShapeDtypeStruct((B,), _F32),
            jax.ShapeDtypeStruct((B,), _F32),
            jax.ShapeDtypeStruct((B,), jnp.int32),
        ),
    )(q_in, keys)


def _rw_body(q_ref, keys_ref, vals_ref, actual_ref, m_ref, l_ref, w_ref,
             slot_ref, retr_ref, nk_ref, nv_ref, *, kd, nblk):
    del nblk
    j = pl.program_id(0)
    q = q_ref[...]
    keys = keys_ref[...]
    vals = vals_ref[...]
    scores_t = _dot(keys, q, (((1,), (1,)))) * (1.0 / (kd ** 0.5))
    m = m_ref[...]
    l = l_ref[...]
    attn_t = jnp.exp(scores_t - m[None, :]) / l[None, :]

    part = _bdot(attn_t, vals, (((0,), (0,))))

    @pl.when(j == 0)
    def _():
        retr_ref[...] = jnp.zeros_like(retr_ref)

    retr_ref[...] += part

    rows = j * SBLK + lax.broadcasted_iota(jnp.int32, (SBLK, 1), 0)
    slot = slot_ref[...]
    w = w_ref[...]
    mwf = jnp.where(rows == slot[None, :], w[None, :], 0.0)
    denom = jnp.sum(mwf, axis=1)
    mw = mwf.astype(jnp.bfloat16)
    numer_v = lax.dot_general(mw, actual_ref[...], ((((1,), (0,))), ((), ())),
                              preferred_element_type=_F32)
    nv_ref[...] = vals * (1.0 - denom)[:, None] + numer_v
    numer_k = lax.dot_general(mw, q.astype(jnp.bfloat16),
                              ((((1,), (0,))), ((), ())),
                              preferred_element_type=_F32)
    nk_ref[...] = keys * (1.0 - denom)[:, None] + numer_k


def _read_write(q, keys, vals, actual, m, l, w, slot, kd):
    nblk = S // SBLK
    full1d = pl.BlockSpec((B,), lambda j: (0,))
    return pl.pallas_call(
        functools.partial(_rw_body, kd=kd, nblk=nblk),
        grid=(nblk,),
        in_specs=[
            pl.BlockSpec((B, kd), lambda j: (0, 0)),
            pl.BlockSpec((SBLK, kd), lambda j: (j, 0)),
            pl.BlockSpec((SBLK, V), lambda j: (j, 0)),
            pl.BlockSpec((B, V), lambda j: (0, 0)),
            full1d, full1d, full1d, full1d,
        ],
        out_specs=(
            pl.BlockSpec((B, V), lambda j: (0, 0)),
            pl.BlockSpec((SBLK, kd), lambda j: (j, 0)),
            pl.BlockSpec((SBLK, V), lambda j: (j, 0)),
        ),
        out_shape=(
            jax.ShapeDtypeStruct((B, V), _F32),
            jax.ShapeDtypeStruct((S, kd), _F32),
            jax.ShapeDtypeStruct((S, V), _F32),
        ),
    )(q, keys, vals, actual, m, l, w, slot)


def _h_body(pc_ref, mc_ref, w1_ref, b1_ref, g_ref, beta_ref, h_ref, *, nk):
    k = pl.program_id(0)
    wblk = w1_ref[...]

    @pl.when(k == 0)
    def _():
        h_ref[...] = jnp.zeros_like(h_ref)

    half = nk // 2

    @pl.when(k < half)
    def _():
        h_ref[...] += _bdot(pc_ref[...], wblk, (((1,), (0,))))

    @pl.when(k >= half)
    def _():
        h_ref[...] += _bdot(mc_ref[...], wblk, (((1,), (0,))))

    @pl.when(k == nk - 1)
    def _():
        h = h_ref[...] + b1_ref[...][None, :]
        mu = jnp.mean(h, axis=1, keepdims=True)
        var = jnp.mean((h - mu) ** 2, axis=1, keepdims=True)
        h = (h - mu) / jnp.sqrt(var + 1e-5) * g_ref[...][None, :] \
            + beta_ref[...][None, :]
        h_ref[...] = 0.5 * h * (1.0 + lax.erf(h * (0.5 ** 0.5)))


def _mlp_h(pair_corr, macro_corr, w1, b1, g, beta):
    nk = (2 * V) // KBLK
    half = nk // 2
    fullv = pl.BlockSpec((V,), lambda k: (0,))
    return pl.pallas_call(
        functools.partial(_h_body, nk=nk),
        grid=(nk,),
        in_specs=[
            pl.BlockSpec((B, KBLK), lambda k: (0, jnp.minimum(k, half - 1))),
            pl.BlockSpec((B, KBLK),
                         lambda k: (0, jnp.maximum(k - half, 0))),
            pl.BlockSpec((KBLK, V), lambda k: (k, 0)),
            fullv, fullv, fullv,
        ],
        out_specs=pl.BlockSpec((B, V), lambda k: (0, 0)),
        out_shape=jax.ShapeDtypeStruct((B, V), _F32),
    )(pair_corr, macro_corr, w1, b1, g, beta)


def _o_body(h_ref, w2_ref, b2_ref, o_ref, *, nk):
    k = pl.program_id(0)

    @pl.when(k == 0)
    def _():
        o_ref[...] = jnp.zeros_like(o_ref)

    o_ref[...] += _bdot(h_ref[...], w2_ref[...], (((1,), (0,))))

    @pl.when(k == nk - 1)
    def _():
        o_ref[...] += b2_ref[...][None, :]


def _mlp_o(h, w2, b2):
    nk = V // KBLK
    return pl.pallas_call(
        functools.partial(_o_body, nk=nk),
        grid=(nk,),
        in_specs=[
            pl.BlockSpec((B, KBLK), lambda k: (0, k)),
            pl.BlockSpec((KBLK, V), lambda k: (k, 0)),
            pl.BlockSpec((V,), lambda k: (0,)),
        ],
        out_specs=pl.BlockSpec((B, V), lambda k: (0, 0)),
        out_shape=jax.ShapeDtypeStruct((B, V), _F32),
    )(h, w2, b2)


def _head_body(ps_ref, f_ref, pw_ref, pb_ref, pg_ref, pbeta_ref, out_ref):
    x1 = ps_ref[...]
    x2 = f_ref[...]
    parts = []
    for i in range(2):
        xs = x1[:, i * D:(i + 1) * D]
        fs = x2[:, i * D:(i + 1) * D]
        pw = pw_ref[i]
        e = _dot(xs, pw[:D], (((1,), (0,)))) \
            + _dot(fs, pw[D:], (((1,), (0,)))) \
            + pb_ref[0, i * D:(i + 1) * D][None, :]
        mu = jnp.mean(e, axis=1, keepdims=True)
        var = jnp.mean((e - mu) ** 2, axis=1, keepdims=True)
        e = (e - mu) / jnp.sqrt(var + 1e-5) \
            * pg_ref[0, i * D:(i + 1) * D][None, :] \
            + pbeta_ref[0, i * D:(i + 1) * D][None, :]
        parts.append(e)
    out_ref[...] = jnp.concatenate(parts, axis=1)


def _heads(ps2d, fused2d, po_W, po_b, po_g, po_beta):
    vec = pl.BlockSpec((1, 2 * D), lambda p: (0, p))
    return pl.pallas_call(
        _head_body,
        grid=(P // 2,),
        in_specs=[
            pl.BlockSpec((B, 2 * D), lambda p: (0, p)),
            pl.BlockSpec((B, 2 * D), lambda p: (0, p)),
            pl.BlockSpec((2, 2 * D, D), lambda p: (p, 0, 0)),
            vec, vec, vec,
        ],
        out_specs=pl.BlockSpec((B, 2 * D), lambda p: (0, p)),
        out_shape=jax.ShapeDtypeStruct((B, P * D), _F32),
    )(ps2d, fused2d, po_W, po_b.reshape(1, P * D), po_g.reshape(1, P * D),
      po_beta.reshape(1, P * D))


def kernel(pair_states, macro_state, W1, b1, ln1_g, ln1_b, W2, b2, po_W,
           po_b, po_g, po_beta, pair_mem_keys, pair_mem_vals,
           macro_mem_keys, macro_mem_vals):
    ps2d = pair_states.reshape(B, V)
    actual_bf = ps2d.astype(jnp.bfloat16)

    pq, pm, plse, pw, pslot = _stats(pair_states, pair_mem_keys, D, True)
    mq, mm, mlse, mw_, mslot = _stats(macro_state, macro_mem_keys, M, False)

    pair_corr, new_pk, new_pv = _read_write(
        pq, pair_mem_keys, pair_mem_vals, actual_bf, pm, plse, pw, pslot, D)
    macro_corr, new_mk, new_mv = _read_write(
        mq, macro_mem_keys, macro_mem_vals, actual_bf, mm, mlse, mw_, mslot, M)

    h = _mlp_h(pair_corr, macro_corr, W1, b1, ln1_g, ln1_b)
    fused2d = _mlp_o(h, W2, b2)

    enriched = _heads(ps2d, fused2d, po_W, po_b, po_g,
                      po_beta).reshape(B, P, D)
    return (enriched, new_pk, new_pv, new_mk, new_mv)


# speedup vs baseline: 1.7442x; 1.0280x over previous
import functools

import jax
import jax.numpy as jnp
from jax import lax
from jax.experimental import pallas as pl

B = 1024
P = 32
D = 64
M = 128
S = 4096
V = 2048

SBLK = 512
KBLK = 1024
BBLK = 256

_F32 = jnp.float32


def _dot(a, b, dims):
    return lax.dot_general(a, b, (dims, ((), ())), preferred_element_type=_F32)


def _bdot(a, b, dims):
    return lax.dot_general(a.astype(jnp.bfloat16), b.astype(jnp.bfloat16),
                           (dims, ((), ())), preferred_element_type=_F32)


def _stats_body(q_ref, k_ref, q_out_ref, m_ref, l_ref, w_ref, slot_ref, *,
                kd, mean_pairs):
    q = q_ref[...]
    if mean_pairs:
        q = jnp.mean(q, axis=1)
    keys = k_ref[...]
    scores = _dot(q, keys, (((1,), (1,)))) * (1.0 / (kd ** 0.5))
    m = jnp.max(scores, axis=1)
    l = jnp.sum(jnp.exp(scores - m[:, None]), axis=1)
    idx = lax.broadcasted_iota(jnp.int32, scores.shape, 1)
    slot = jnp.min(jnp.where(scores == m[:, None], idx, S), axis=1)
    surprise = 1.0 - 1.0 / l
    w = 0.1 * jax.nn.sigmoid(surprise)
    q_out_ref[...] = q
    m_ref[...] = m
    l_ref[...] = l
    w_ref[...] = w
    slot_ref[...] = slot.astype(jnp.int32)


def _stats(q_in, keys, kd, mean_pairs):
    qd = q_in.shape[-1] if not mean_pairs else q_in.shape[-1]
    return pl.pallas_call(
        functools.partial(_stats_body, kd=kd, mean_pairs=mean_pairs),
        out_shape=(
            jax.ShapeDtypeStruct((B, qd), _F32),
            jax.ShapeDtypeStruct((B,), _F32),
            jax.ShapeDtypeStruct((B,), _F32),
            jax.ShapeDtypeStruct((B,), _F32),
            jax.ShapeDtypeStruct((B,), jnp.int32),
        ),
    )(q_in, keys)


def _rw_body(q_ref, keys_ref, vals_ref, actual_ref, m_ref, l_ref, w_ref,
             slot_ref, retr_ref, nk_ref, nv_ref, *, kd, nblk):
    del nblk
    j = pl.program_id(0)
    q = q_ref[...]
    keys = keys_ref[...]
    vals = vals_ref[...]
    scores_t = _dot(keys, q, (((1,), (1,)))) * (1.0 / (kd ** 0.5))
    m = m_ref[...]
    l = l_ref[...]
    attn_t = jnp.exp(scores_t - m[None, :]) / l[None, :]

    part = _dot(attn_t, vals, (((0,), (0,))))

    @pl.when(j == 0)
    def _():
        retr_ref[...] = part

    @pl.when(j > 0)
    def _():
        retr_ref[...] += part

    rows = j * SBLK + lax.broadcasted_iota(jnp.int32, (SBLK, 1), 0)
    slot = slot_ref[...]
    w = w_ref[...]
    mwf = jnp.where(rows == slot[None, :], w[None, :], 0.0)
    denom = jnp.sum(mwf, axis=1)
    mw = mwf.astype(jnp.bfloat16)
    numer_v = lax.dot_general(mw, actual_ref[...], ((((1,), (0,))), ((), ())),
                              preferred_element_type=_F32)
    nv_ref[...] = vals * (1.0 - denom)[:, None] + numer_v
    numer_k = lax.dot_general(mw, q.astype(jnp.bfloat16),
                              ((((1,), (0,))), ((), ())),
                              preferred_element_type=_F32)
    nk_ref[...] = keys * (1.0 - denom)[:, None] + numer_k


def _read_write(q, keys, vals, actual, m, l, w, slot, kd):
    nblk = S // SBLK
    full1d = pl.BlockSpec((B,), lambda j: (0,))
    return pl.pallas_call(
        functools.partial(_rw_body, kd=kd, nblk=nblk),
        grid=(nblk,),
        in_specs=[
            pl.BlockSpec((B, kd), lambda j: (0, 0)),
            pl.BlockSpec((SBLK, kd), lambda j: (j, 0)),
            pl.BlockSpec((SBLK, V), lambda j: (j, 0)),
            pl.BlockSpec((B, V), lambda j: (0, 0)),
            full1d, full1d, full1d, full1d,
        ],
        out_specs=(
            pl.BlockSpec((B, V), lambda j: (0, 0)),
            pl.BlockSpec((SBLK, kd), lambda j: (j, 0)),
            pl.BlockSpec((SBLK, V), lambda j: (j, 0)),
        ),
        out_shape=(
            jax.ShapeDtypeStruct((B, V), _F32),
            jax.ShapeDtypeStruct((S, kd), _F32),
            jax.ShapeDtypeStruct((S, V), _F32),
        ),
    )(q, keys, vals, actual, m, l, w, slot)


def _h_body(pc_ref, mc_ref, w1_ref, b1_ref, g_ref, beta_ref, h_ref, *, nk):
    k = pl.program_id(0)
    wblk = w1_ref[...]

    @pl.when(k == 0)
    def _():
        h_ref[...] = jnp.zeros_like(h_ref)

    half = nk // 2

    @pl.when(k < half)
    def _():
        h_ref[...] += _bdot(pc_ref[...], wblk, (((1,), (0,))))

    @pl.when(k >= half)
    def _():
        h_ref[...] += _bdot(mc_ref[...], wblk, (((1,), (0,))))

    @pl.when(k == nk - 1)
    def _():
        h = h_ref[...] + b1_ref[...][None, :]
        mu = jnp.mean(h, axis=1, keepdims=True)
        var = jnp.mean((h - mu) ** 2, axis=1, keepdims=True)
        h = (h - mu) / jnp.sqrt(var + 1e-5) * g_ref[...][None, :] \
            + beta_ref[...][None, :]
        h_ref[...] = 0.5 * h * (1.0 + lax.erf(h * (0.5 ** 0.5)))


def _mlp_h(pair_corr, macro_corr, w1, b1, g, beta):
    nk = (2 * V) // KBLK
    half = nk // 2
    fullv = pl.BlockSpec((V,), lambda k: (0,))
    return pl.pallas_call(
        functools.partial(_h_body, nk=nk),
        grid=(nk,),
        in_specs=[
            pl.BlockSpec((B, KBLK), lambda k: (0, jnp.minimum(k, half - 1))),
            pl.BlockSpec((B, KBLK),
                         lambda k: (0, jnp.maximum(k - half, 0))),
            pl.BlockSpec((KBLK, V), lambda k: (k, 0)),
            fullv, fullv, fullv,
        ],
        out_specs=pl.BlockSpec((B, V), lambda k: (0, 0)),
        out_shape=jax.ShapeDtypeStruct((B, V), _F32),
    )(pair_corr, macro_corr, w1, b1, g, beta)


def _o_body(h_ref, w2_ref, b2_ref, o_ref, *, nk):
    k = pl.program_id(0)

    @pl.when(k == 0)
    def _():
        o_ref[...] = jnp.zeros_like(o_ref)

    o_ref[...] += _bdot(h_ref[...], w2_ref[...], (((1,), (0,))))

    @pl.when(k == nk - 1)
    def _():
        o_ref[...] += b2_ref[...][None, :]


def _mlp_o(h, w2, b2):
    nk = V // KBLK
    return pl.pallas_call(
        functools.partial(_o_body, nk=nk),
        grid=(nk,),
        in_specs=[
            pl.BlockSpec((B, KBLK), lambda k: (0, k)),
            pl.BlockSpec((KBLK, V), lambda k: (k, 0)),
            pl.BlockSpec((V,), lambda k: (0,)),
        ],
        out_specs=pl.BlockSpec((B, V), lambda k: (0, 0)),
        out_shape=jax.ShapeDtypeStruct((B, V), _F32),
    )(h, w2, b2)


def _head_body(ps_ref, f_ref, pw_ref, pb_ref, pg_ref, pbeta_ref, out_ref):
    x1 = ps_ref[...]
    x2 = f_ref[...]
    parts = []
    for i in range(2):
        xs = x1[:, i * D:(i + 1) * D]
        fs = x2[:, i * D:(i + 1) * D]
        pw = pw_ref[i]
        e = _dot(xs, pw[:D], (((1,), (0,)))) \
            + _dot(fs, pw[D:], (((1,), (0,)))) \
            + pb_ref[0, i * D:(i + 1) * D][None, :]
        mu = jnp.mean(e, axis=1, keepdims=True)
        var = jnp.mean((e - mu) ** 2, axis=1, keepdims=True)
        e = (e - mu) / jnp.sqrt(var + 1e-5) \
            * pg_ref[0, i * D:(i + 1) * D][None, :] \
            + pbeta_ref[0, i * D:(i + 1) * D][None, :]
        parts.append(e)
    out_ref[...] = jnp.concatenate(parts, axis=1)


def _heads(ps2d, fused2d, po_W, po_b, po_g, po_beta):
    vec = pl.BlockSpec((1, 2 * D), lambda p: (0, p))
    return pl.pallas_call(
        _head_body,
        grid=(P // 2,),
        in_specs=[
            pl.BlockSpec((B, 2 * D), lambda p: (0, p)),
            pl.BlockSpec((B, 2 * D), lambda p: (0, p)),
            pl.BlockSpec((2, 2 * D, D), lambda p: (p, 0, 0)),
            vec, vec, vec,
        ],
        out_specs=pl.BlockSpec((B, 2 * D), lambda p: (0, p)),
        out_shape=jax.ShapeDtypeStruct((B, P * D), _F32),
    )(ps2d, fused2d, po_W, po_b.reshape(1, P * D), po_g.reshape(1, P * D),
      po_beta.reshape(1, P * D))


def kernel(pair_states, macro_state, W1, b1, ln1_g, ln1_b, W2, b2, po_W,
           po_b, po_g, po_beta, pair_mem_keys, pair_mem_vals,
           macro_mem_keys, macro_mem_vals):
    ps2d = pair_states.reshape(B, V)
    actual_bf = ps2d.astype(jnp.bfloat16)

    pq, pm, plse, pw, pslot = _stats(pair_states, pair_mem_keys, D, True)
    mq, mm, mlse, mw_, mslot = _stats(macro_state, macro_mem_keys, M, False)

    pair_corr, new_pk, new_pv = _read_write(
        pq, pair_mem_keys, pair_mem_vals, actual_bf, pm, plse, pw, pslot, D)
    macro_corr, new_mk, new_mv = _read_write(
        mq, macro_mem_keys, macro_mem_vals, actual_bf, mm, mlse, mw_, mslot, M)

    h = _mlp_h(pair_corr, macro_corr, W1, b1, ln1_g, ln1_b)
    fused2d = _mlp_o(h, W2, b2)

    enriched = _heads(ps2d, fused2d, po_W, po_b, po_g,
                      po_beta).reshape(B, P, D)
    return (enriched, new_pk, new_pv, new_mk, new_mv)
